# baseline (speedup 1.0000x reference)
import functools

import jax
import jax.numpy as jnp
from jax import lax
from jax.experimental import pallas as pl
from jax.experimental.pallas import tpu as pltpu
from jax.experimental.pallas import tpu_sc as plsc

N = 10000
D = 128
E = 320000
NDP = 320
CAP = 12288
HASH = 16384
CHUNK = 6400
RB = 128
LN2 = 0.6931471805599453
GAMMA = 0.5


_GDN = lax.GatherDimensionNumbers(
    offset_dims=(), collapsed_slice_dims=(0,), start_index_map=(0,))


def _bcast_lane(vec, l):
    idx = jnp.full((16, 1), l, jnp.int32)
    return lax.gather(vec, idx, _GDN, slice_sizes=(1,),
                      mode=lax.GatherScatterMode.PROMISE_IN_BOUNDS)


def _sigmoid(x):
    return 1.0 / (1.0 + jnp.exp(-x))


def _rsqrt_newton(x):
    bits = plsc.bitcast(x, jnp.int32)
    y = plsc.bitcast(jnp.int32(0x5F3759DF) - lax.shift_right_arithmetic(bits, 1),
                     jnp.float32)
    for _ in range(3):
        y = y * (1.5 - 0.5 * x * y * y)
    return y


def _w_occ(cf):
    u = cf + 1.0
    bits = plsc.bitcast(u, jnp.int32)
    e = lax.shift_right_arithmetic(bits, 23) - 127
    m = plsc.bitcast((bits & 0x007FFFFF) | 0x3F800000, jnp.float32)
    big = m > 1.5
    m = jnp.where(big, m * 0.5, m)
    ef = (e + big.astype(jnp.int32)).astype(jnp.float32)
    t = (m - 1.0) / (m + 1.0)
    t2 = t * t
    p = t * (2.0 + t2 * (2.0 / 3.0 + t2 * (0.4 + t2 * (2.0 / 7.0))))
    lnu = ef * LN2 + p
    return lnu / jnp.maximum(cf, 1.0)


def _sc_mesh():
    return plsc.VectorSubcoreMesh(core_axis_name="c", subcore_axis_name="s")


def _num_tiles():
    info = plsc.get_sparse_core_info()
    return info.num_cores * info.num_subcores


def _edge_prep(srcs, dsts):
    nt = _num_tiles()
    ns = plsc.get_sparse_core_info().num_subcores
    npad = nt * NDP

    def body(srcs_hbm, dsts_hbm, es_hbm, ed_hbm, ew_hbm, dinv_hbm, en_hbm,
             cbs, cbd, ces, ced, cew, ka, pa, tk, tc, dg, dv, env):
        wid = lax.axis_index("c") * ns + lax.axis_index("s")
        lo = wid * NDP
        iota = lax.iota(jnp.int32, 16)
        zf = jnp.zeros((16,), jnp.float32)

        def initb(i, _):
            pos = i * 16 + iota
            ces[pl.ds(i * 16, 16)] = pos & 8191
            ced[pl.ds(i * 16, 16)] = jnp.zeros((16,), jnp.int32) + lo
            cew[pl.ds(i * 16, 16)] = zf
            return 0

        lax.fori_loop(0, CAP // 16, initb, 0)

        def inith(i, _):
            tk[pl.ds(i * 16, 16)] = jnp.full((16,), -1, jnp.int32)
            tc[pl.ds(i * 16, 16)] = jnp.zeros((16,), jnp.int32)
            return 0

        lax.fori_loop(0, HASH // 16, inith, 0)

        def initd(i, _):
            dg[pl.ds(i * 16, 16)] = zf
            return 0

        lax.fori_loop(0, (16 * NDP) // 16, initd, 0)

        def chunk_body(c, off):
            pltpu.sync_copy(srcs_hbm.at[pl.ds(c * CHUNK, CHUNK)], cbs)
            pltpu.sync_copy(dsts_hbm.at[pl.ds(c * CHUNK, CHUNK)], cbd)

            def vbody(i, off):
                s = cbs[pl.ds(i * 16, 16)]
                d = cbd[pl.ds(i * 16, 16)]
                m = (d >= lo) & (d < lo + NDP)
                offc = jnp.minimum(off, CAP - 16)
                plsc.store_compressed(ces.at[pl.ds(offc, 16)], s, mask=m)
                plsc.store_compressed(ced.at[pl.ds(offc, 16)], d, mask=m)
                pc = plsc.all_reduce_population_count(m)
                return jnp.minimum(off + pc[0], jnp.int32(CAP - 16))

            return lax.fori_loop(0, CHUNK // 16, vbody, off)

        ne = lax.fori_loop(0, E // CHUNK, chunk_body, jnp.int32(0))
        nv16 = (ne + 15) // 16

        def kinit(i, _):
            pos = i * 16 + iota
            s = ces[pl.ds(i * 16, 16)]
            d = ced[pl.ds(i * 16, 16)]
            ka[pl.ds(i * 16, 16)] = s * NDP + (d - lo)
            pa[pl.ds(i * 16, 16)] = pos
            return 0

        lax.fori_loop(0, nv16, kinit, 0)

        def rcond(st):
            return st[0] > 0

        def rbody(st):
            na, r = st
            mult = jnp.int32(0x9E3779B1 - (1 << 32)) + r * jnp.int32(
                0x85EBCA6A - (1 << 32))
            nv = (na + 15) // 16

            def hash_of(k):
                return lax.shift_right_logical(k * mult, 17) & (HASH - 1)

            def la(i, _):
                valid = (i * 16 + iota) < na
                k = ka[pl.ds(i * 16, 16)]
                plsc.store_scatter(tk, [hash_of(k)], k, mask=valid)
                return 0

            lax.fori_loop(0, nv, la, 0)

            def lb(i, _):
                valid = (i * 16 + iota) < na
                k = ka[pl.ds(i * 16, 16)]
                h = hash_of(k)
                t = plsc.load_gather(tk, [h], mask=valid)
                win = valid & (t == k)
                cnt, last = plsc.scan_count(k, win)
                plsc.addupdate_scatter(tc, [h], cnt, mask=last)
                return 0

            lax.fori_loop(0, nv, lb, 0)

            def lc(i, na2):
                valid = (i * 16 + iota) < na
                k = ka[pl.ds(i * 16, 16)]
                p = pa[pl.ds(i * 16, 16)]
                h = hash_of(k)
                t = plsc.load_gather(tk, [h], mask=valid)
                win = valid & (t == k)
                c = plsc.load_gather(tc, [h], mask=win)
                w = _w_occ(c.astype(jnp.float32))
                plsc.store_scatter(cew, [p], w, mask=win)
                lose = valid & jnp.logical_not(win)
                na2c = jnp.minimum(na2, jnp.int32(CAP - 16))
                plsc.store_compressed(ka.at[pl.ds(na2c, 16)], k, mask=lose)
                plsc.store_compressed(pa.at[pl.ds(na2c, 16)], p, mask=lose)
                pc = plsc.all_reduce_population_count(lose)
                return na2 + pc[0]

            na2 = lax.fori_loop(0, nv, lc, jnp.int32(0))
            lax.fori_loop(0, HASH // 16, inith, 0)
            return (na2, r + 1)

        lax.while_loop(rcond, rbody, (ne, jnp.int32(0)))

        def dbody(i, _):
            d = ced[pl.ds(i * 16, 16)]
            w = cew[pl.ds(i * 16, 16)]
            idx = iota * NDP + (d - lo)
            plsc.addupdate_scatter(dg, [idx], w)
            return 0

        lax.fori_loop(0, nv16, dbody, 0)

        def fbody(m, _):
            acc = jnp.full((16,), 1.0, jnp.float32)
            for l in range(16):
                acc = acc + dg[pl.ds(l * NDP + m * 16, 16)]
            dv[pl.ds(m * 16, 16)] = _rsqrt_newton(acc)
            return 0

        lax.fori_loop(0, NDP // 16, fbody, 0)

        pltpu.sync_copy(ces, es_hbm.at[wid])
        pltpu.sync_copy(ced, ed_hbm.at[wid])
        pltpu.sync_copy(cew, ew_hbm.at[wid])
        pltpu.sync_copy(dv, dinv_hbm.at[wid])
        env[pl.ds(0, 16)] = jnp.zeros((16,), jnp.int32) + ne
        pltpu.sync_copy(env, en_hbm.at[wid])

    f = pl.kernel(
        body,
        out_type=(
            jax.ShapeDtypeStruct((nt, CAP), jnp.int32),
            jax.ShapeDtypeStruct((nt, CAP), jnp.int32),
            jax.ShapeDtypeStruct((nt, CAP), jnp.float32),
            jax.ShapeDtypeStruct((nt, NDP), jnp.float32),
            jax.ShapeDtypeStruct((nt, 16), jnp.int32),
        ),
        mesh=_sc_mesh(),
        scratch_types=[
            pltpu.VMEM((CHUNK,), jnp.int32),
            pltpu.VMEM((CHUNK,), jnp.int32),
            pltpu.VMEM((CAP,), jnp.int32),
            pltpu.VMEM((CAP,), jnp.int32),
            pltpu.VMEM((CAP,), jnp.float32),
            pltpu.VMEM((CAP,), jnp.int32),
            pltpu.VMEM((CAP,), jnp.int32),
            pltpu.VMEM((HASH,), jnp.int32),
            pltpu.VMEM((HASH,), jnp.int32),
            pltpu.VMEM((16 * NDP,), jnp.float32),
            pltpu.VMEM((NDP,), jnp.float32),
            pltpu.VMEM((16,), jnp.int32),
        ],
        compiler_params=pltpu.CompilerParams(needs_layout_passes=False),
    )
    es, ed, ew, dinv, en = f(srcs, dsts)
    return es, ed, ew, dinv.reshape(npad), en


def _layer(es, ed, ew, dinv, en, sup, y1):
    nt = _num_tiles()
    ns = plsc.get_sparse_core_info().num_subcores
    npad = nt * NDP
    with_comb = y1 is not None

    def body(*refs):
        if with_comb:
            (es_hbm, ed_hbm, ew_hbm, dinv_hbm, en_hbm, sup_hbm, y1_hbm,
             y_hbm, ies, ied, iew, dib, acc, rows0, rows1, tmps, tmp1, env,
             sem0, sem1) = refs
        else:
            (es_hbm, ed_hbm, ew_hbm, dinv_hbm, en_hbm, sup_hbm,
             y_hbm, ies, ied, iew, dib, acc, rows0, rows1, tmps, tmp1, env,
             sem0, sem1) = refs
        wid = lax.axis_index("c") * ns + lax.axis_index("s")
        lo = wid * NDP
        iota = lax.iota(jnp.int32, 16)
        zf = jnp.zeros((16,), jnp.float32)

        pltpu.sync_copy(es_hbm.at[wid], ies)
        pltpu.sync_copy(ed_hbm.at[wid], ied)
        pltpu.sync_copy(ew_hbm.at[wid], iew)
        pltpu.sync_copy(dinv_hbm, dib)
        pltpu.sync_copy(en_hbm.at[wid], env)
        ne = env[pl.ds(0, 16)][0]
        nv16 = (ne + 15) // 16

        def zbody2(i, _):
            for j in range(D // 16):
                acc[i, pl.ds(j * 16, 16)] = zf
            return 0

        lax.fori_loop(0, NDP, zbody2, 0)

        def cbody(i, _):
            s = ies[pl.ds(i * 16, 16)]
            d = ied[pl.ds(i * 16, 16)]
            w = iew[pl.ds(i * 16, 16)]
            ds_ = plsc.load_gather(dib, [s])
            dd = plsc.load_gather(dib, [d])
            iew[pl.ds(i * 16, 16)] = w * ds_ * dd
            return 0

        lax.fori_loop(0, nv16, cbody, 0)

        nct = 2 * ((ne + 2 * RB - 1) // (2 * RB))

        def fire(c, buf, sem):
            pltpu.async_copy(sup_hbm.at[pl.ds(0, RB)], buf, sem)

        def drain(c, buf, sem):
            pltpu.make_async_copy(sup_hbm.at[pl.ds(0, RB)], buf,
                                  sem).wait()

        zi = jnp.zeros((16,), jnp.int32)

        def process(c, buf):
            ilv = 4

            def ebody(g, _):
                eb = c * RB + g * 16
                dvec = ied[pl.ds(eb, 16)] - lo
                cvec = iew[pl.ds(eb, 16)]
                for l0 in range(0, 16, ilv):
                    cfb = [_bcast_lane(cvec, l0 + u) for u in range(ilv)]
                    dlb = [_bcast_lane(dvec, l0 + u) for u in range(ilv)]
                    for j in range(D // 16):
                        col = j * 16 + iota
                        for u in range(ilv):
                            v = cfb[u] * buf[g * 16 + l0 + u,
                                             pl.ds(j * 16, 16)]
                            plsc.addupdate_scatter(acc, [dlb[u], col], v)
                return 0

            lax.fori_loop(0, RB // 16, ebody, 0)

        @pl.when(nct > 0)
        def _():
            fire(0, rows0, sem0)

        def ch2(i, _):
            c0 = 2 * i
            fire(c0 + 1, rows1, sem1)
            drain(c0, rows0, sem0)
            process(c0, rows0)

            @pl.when(c0 + 2 < nct)
            def _():
                fire(c0 + 2, rows0, sem0)

            drain(c0 + 1, rows1, sem1)
            process(c0 + 1, rows1)
            return 0

        lax.fori_loop(0, nct // 2, ch2, 0)

        def sbody(m, _):
            pltpu.sync_copy(sup_hbm.at[pl.ds(lo + m * 16, 16)], tmps)
            if with_comb:
                pltpu.sync_copy(y1_hbm.at[pl.ds(lo + m * 16, 16)], tmp1)
            rr = lo + m * 16 + iota
            dvv = plsc.load_gather(dib, [rr])
            d2 = dvv * dvv
            for l in range(16):
                rloc = m * 16 + l
                d2l = d2[l]
                for j in range(D // 16):
                    a = acc[rloc, pl.ds(j * 16, 16)]
                    sv = tmps[l, pl.ds(j * 16, 16)]
                    out = _sigmoid(a + d2l * sv)
                    if with_comb:
                        out = GAMMA * tmp1[l, pl.ds(j * 16, 16)] + \
                            (1.0 - GAMMA) * out
                    acc[rloc, pl.ds(j * 16, 16)] = out
            return 0

        lax.fori_loop(0, NDP // 16, sbody, 0)

        pltpu.sync_copy(acc, y_hbm.at[pl.ds(lo, NDP)])

    ins = (es, ed, ew, dinv, en, sup) + ((y1,) if with_comb else ())
    f = pl.kernel(
        body,
        out_type=jax.ShapeDtypeStruct((npad, D), jnp.float32),
        mesh=_sc_mesh(),
        scratch_types=[
            pltpu.VMEM((CAP,), jnp.int32),
            pltpu.VMEM((CAP,), jnp.int32),
            pltpu.VMEM((CAP,), jnp.float32),
            pltpu.VMEM((npad,), jnp.float32),
            pltpu.VMEM((NDP, D), jnp.float32),
            pltpu.VMEM((RB, D), jnp.float32),
            pltpu.VMEM((RB, D), jnp.float32),
            pltpu.VMEM((16, D), jnp.float32),
            pltpu.VMEM((16, D), jnp.float32),
            pltpu.VMEM((16,), jnp.int32),
            pltpu.SemaphoreType.DMA,
            pltpu.SemaphoreType.DMA,
        ],
        compiler_params=pltpu.CompilerParams(needs_layout_passes=False),
    )
    return f(*ins)


def _matmul(x, w):
    bm = 1024
    npad = x.shape[0]

    def body(x_ref, w_ref, o_ref):
        o_ref[...] = jnp.dot(x_ref[...], w_ref[...],
                             preferred_element_type=jnp.float32)

    return pl.pallas_call(
        body,
        grid=(npad // bm,),
        in_specs=[
            pl.BlockSpec((bm, D), lambda i: (i, 0)),
            pl.BlockSpec((D, D), lambda i: (0, 0)),
        ],
        out_specs=pl.BlockSpec((bm, D), lambda i: (i, 0)),
        out_shape=jax.ShapeDtypeStruct((npad, D), jnp.float32),
    )(x, w)


def _gated_combine(xp, comb, wg):
    bm = 1024
    npad = xp.shape[0]

    def body(x_ref, c_ref, w_ref, o_ref):
        c = c_ref[...]
        g = jnp.dot(c, w_ref[...], preferred_element_type=jnp.float32)
        o_ref[...] = x_ref[...] + jax.nn.sigmoid(g) * c

    return pl.pallas_call(
        body,
        grid=(npad // bm,),
        in_specs=[
            pl.BlockSpec((bm, D), lambda i: (i, 0)),
            pl.BlockSpec((bm, D), lambda i: (i, 0)),
            pl.BlockSpec((D, D), lambda i: (0, 0)),
        ],
        out_specs=pl.BlockSpec((bm, D), lambda i: (i, 0)),
        out_shape=jax.ShapeDtypeStruct((npad, D), jnp.float32),
    )(xp, comb, wg)


def kernel(node_embeddings, cached_edges, weight1, weight_gate):
    nt = _num_tiles()
    npad = nt * NDP
    srcs = cached_edges[:, 0].astype(jnp.int32)
    dsts = cached_edges[:, 1].astype(jnp.int32)
    xp = jnp.zeros((npad, D), jnp.float32).at[:N].set(node_embeddings)

    es, ed, ew, dinv, en = _edge_prep(srcs, dsts)
    sup1 = _matmul(xp, weight1)
    y1 = _layer(es, ed, ew, dinv, en, sup1, None)
    sup2 = _matmul(y1, weight1)
    comb = _layer(es, ed, ew, dinv, en, sup2, y1)
    out = _gated_combine(xp, comb, weight_gate)
    return out[:N]


# speedup vs baseline: 1.0032x; 1.0032x over previous
import functools

import jax
import jax.numpy as jnp
from jax import lax
from jax.experimental import pallas as pl
from jax.experimental.pallas import tpu as pltpu
from jax.experimental.pallas import tpu_sc as plsc

N = 10000
D = 128
E = 320000
NDP = 320
CAP = 12288
HASH = 16384
CHUNK = 6400
RB = 128
LN2 = 0.6931471805599453
GAMMA = 0.5


_GDN = lax.GatherDimensionNumbers(
    offset_dims=(), collapsed_slice_dims=(0,), start_index_map=(0,))


def _bcast_lane(vec, l):
    idx = jnp.full((16, 1), l, jnp.int32)
    return lax.gather(vec, idx, _GDN, slice_sizes=(1,),
                      mode=lax.GatherScatterMode.PROMISE_IN_BOUNDS)


def _sigmoid(x):
    return 1.0 / (1.0 + jnp.exp(-x))


def _rsqrt_newton(x):
    bits = plsc.bitcast(x, jnp.int32)
    y = plsc.bitcast(jnp.int32(0x5F3759DF) - lax.shift_right_arithmetic(bits, 1),
                     jnp.float32)
    for _ in range(3):
        y = y * (1.5 - 0.5 * x * y * y)
    return y


def _w_occ(cf):
    u = cf + 1.0
    bits = plsc.bitcast(u, jnp.int32)
    e = lax.shift_right_arithmetic(bits, 23) - 127
    m = plsc.bitcast((bits & 0x007FFFFF) | 0x3F800000, jnp.float32)
    big = m > 1.5
    m = jnp.where(big, m * 0.5, m)
    ef = (e + big.astype(jnp.int32)).astype(jnp.float32)
    t = (m - 1.0) / (m + 1.0)
    t2 = t * t
    p = t * (2.0 + t2 * (2.0 / 3.0 + t2 * (0.4 + t2 * (2.0 / 7.0))))
    lnu = ef * LN2 + p
    return lnu / jnp.maximum(cf, 1.0)


def _sc_mesh():
    return plsc.VectorSubcoreMesh(core_axis_name="c", subcore_axis_name="s")


def _num_tiles():
    info = plsc.get_sparse_core_info()
    return info.num_cores * info.num_subcores


def _edge_prep(srcs, dsts):
    nt = _num_tiles()
    ns = plsc.get_sparse_core_info().num_subcores
    npad = nt * NDP

    def body(srcs_hbm, dsts_hbm, es_hbm, ed_hbm, ew_hbm, dinv_hbm, en_hbm,
             cbs, cbd, ces, ced, cew, ka, pa, tk, tc, dg, dv, env):
        wid = lax.axis_index("c") * ns + lax.axis_index("s")
        lo = wid * NDP
        iota = lax.iota(jnp.int32, 16)
        zf = jnp.zeros((16,), jnp.float32)

        def initb(i, _):
            pos = i * 16 + iota
            ces[pl.ds(i * 16, 16)] = pos & 8191
            ced[pl.ds(i * 16, 16)] = jnp.zeros((16,), jnp.int32) + lo
            cew[pl.ds(i * 16, 16)] = zf
            return 0

        lax.fori_loop(0, CAP // 16, initb, 0)

        def inith(i, _):
            tk[pl.ds(i * 16, 16)] = jnp.full((16,), -1, jnp.int32)
            tc[pl.ds(i * 16, 16)] = jnp.zeros((16,), jnp.int32)
            return 0

        lax.fori_loop(0, HASH // 16, inith, 0)

        def initd(i, _):
            dg[pl.ds(i * 16, 16)] = zf
            return 0

        lax.fori_loop(0, (16 * NDP) // 16, initd, 0)

        def chunk_body(c, off):
            pltpu.sync_copy(srcs_hbm.at[pl.ds(c * CHUNK, CHUNK)], cbs)
            pltpu.sync_copy(dsts_hbm.at[pl.ds(c * CHUNK, CHUNK)], cbd)

            def vbody(i, off):
                s = cbs[pl.ds(i * 16, 16)]
                d = cbd[pl.ds(i * 16, 16)]
                m = (d >= lo) & (d < lo + NDP)
                offc = jnp.minimum(off, CAP - 16)
                plsc.store_compressed(ces.at[pl.ds(offc, 16)], s, mask=m)
                plsc.store_compressed(ced.at[pl.ds(offc, 16)], d, mask=m)
                pc = plsc.all_reduce_population_count(m)
                return jnp.minimum(off + pc[0], jnp.int32(CAP - 16))

            return lax.fori_loop(0, CHUNK // 16, vbody, off)

        ne = lax.fori_loop(0, E // CHUNK, chunk_body, jnp.int32(0))
        nv16 = (ne + 15) // 16

        def kinit(i, _):
            pos = i * 16 + iota
            s = ces[pl.ds(i * 16, 16)]
            d = ced[pl.ds(i * 16, 16)]
            ka[pl.ds(i * 16, 16)] = s * NDP + (d - lo)
            pa[pl.ds(i * 16, 16)] = pos
            return 0

        lax.fori_loop(0, nv16, kinit, 0)

        def rcond(st):
            return st[0] > 0

        def rbody(st):
            na, r = st
            mult = jnp.int32(0x9E3779B1 - (1 << 32)) + r * jnp.int32(
                0x85EBCA6A - (1 << 32))
            nv = (na + 15) // 16

            def hash_of(k):
                return lax.shift_right_logical(k * mult, 17) & (HASH - 1)

            def la(i, _):
                valid = (i * 16 + iota) < na
                k = ka[pl.ds(i * 16, 16)]
                plsc.store_scatter(tk, [hash_of(k)], k, mask=valid)
                return 0

            lax.fori_loop(0, nv, la, 0)

            def lb(i, _):
                valid = (i * 16 + iota) < na
                k = ka[pl.ds(i * 16, 16)]
                h = hash_of(k)
                t = plsc.load_gather(tk, [h], mask=valid)
                win = valid & (t == k)
                cnt, last = plsc.scan_count(k, win)
                plsc.addupdate_scatter(tc, [h], cnt, mask=last)
                return 0

            lax.fori_loop(0, nv, lb, 0)

            def lc(i, na2):
                valid = (i * 16 + iota) < na
                k = ka[pl.ds(i * 16, 16)]
                p = pa[pl.ds(i * 16, 16)]
                h = hash_of(k)
                t = plsc.load_gather(tk, [h], mask=valid)
                win = valid & (t == k)
                c = plsc.load_gather(tc, [h], mask=win)
                w = _w_occ(c.astype(jnp.float32))
                plsc.store_scatter(cew, [p], w, mask=win)
                lose = valid & jnp.logical_not(win)
                na2c = jnp.minimum(na2, jnp.int32(CAP - 16))
                plsc.store_compressed(ka.at[pl.ds(na2c, 16)], k, mask=lose)
                plsc.store_compressed(pa.at[pl.ds(na2c, 16)], p, mask=lose)
                pc = plsc.all_reduce_population_count(lose)
                return na2 + pc[0]

            na2 = lax.fori_loop(0, nv, lc, jnp.int32(0))
            lax.fori_loop(0, HASH // 16, inith, 0)
            return (na2, r + 1)

        lax.while_loop(rcond, rbody, (ne, jnp.int32(0)))

        def dbody(i, _):
            d = ced[pl.ds(i * 16, 16)]
            w = cew[pl.ds(i * 16, 16)]
            idx = iota * NDP + (d - lo)
            plsc.addupdate_scatter(dg, [idx], w)
            return 0

        lax.fori_loop(0, nv16, dbody, 0)

        def fbody(m, _):
            acc = jnp.full((16,), 1.0, jnp.float32)
            for l in range(16):
                acc = acc + dg[pl.ds(l * NDP + m * 16, 16)]
            dv[pl.ds(m * 16, 16)] = _rsqrt_newton(acc)
            return 0

        lax.fori_loop(0, NDP // 16, fbody, 0)

        pltpu.sync_copy(ces, es_hbm.at[wid])
        pltpu.sync_copy(ced, ed_hbm.at[wid])
        pltpu.sync_copy(cew, ew_hbm.at[wid])
        pltpu.sync_copy(dv, dinv_hbm.at[wid])
        env[pl.ds(0, 16)] = jnp.zeros((16,), jnp.int32) + ne
        pltpu.sync_copy(env, en_hbm.at[wid])

    f = pl.kernel(
        body,
        out_type=(
            jax.ShapeDtypeStruct((nt, CAP), jnp.int32),
            jax.ShapeDtypeStruct((nt, CAP), jnp.int32),
            jax.ShapeDtypeStruct((nt, CAP), jnp.float32),
            jax.ShapeDtypeStruct((nt, NDP), jnp.float32),
            jax.ShapeDtypeStruct((nt, 16), jnp.int32),
        ),
        mesh=_sc_mesh(),
        scratch_types=[
            pltpu.VMEM((CHUNK,), jnp.int32),
            pltpu.VMEM((CHUNK,), jnp.int32),
            pltpu.VMEM((CAP,), jnp.int32),
            pltpu.VMEM((CAP,), jnp.int32),
            pltpu.VMEM((CAP,), jnp.float32),
            pltpu.VMEM((CAP,), jnp.int32),
            pltpu.VMEM((CAP,), jnp.int32),
            pltpu.VMEM((HASH,), jnp.int32),
            pltpu.VMEM((HASH,), jnp.int32),
            pltpu.VMEM((16 * NDP,), jnp.float32),
            pltpu.VMEM((NDP,), jnp.float32),
            pltpu.VMEM((16,), jnp.int32),
        ],
        compiler_params=pltpu.CompilerParams(needs_layout_passes=False),
    )
    es, ed, ew, dinv, en = f(srcs, dsts)
    return es, ed, ew, dinv.reshape(npad), en


def _layer(es, ed, ew, dinv, en, sup, y1):
    nt = _num_tiles()
    ns = plsc.get_sparse_core_info().num_subcores
    npad = nt * NDP
    with_comb = y1 is not None

    def body(*refs):
        if with_comb:
            (es_hbm, ed_hbm, ew_hbm, dinv_hbm, en_hbm, sup_hbm, y1_hbm,
             y_hbm, ies, ied, iew, dib, acc, rows0, rows1, tmps, tmp1, env,
             sem0, sem1) = refs
        else:
            (es_hbm, ed_hbm, ew_hbm, dinv_hbm, en_hbm, sup_hbm,
             y_hbm, ies, ied, iew, dib, acc, rows0, rows1, tmps, tmp1, env,
             sem0, sem1) = refs
        wid = lax.axis_index("c") * ns + lax.axis_index("s")
        lo = wid * NDP
        iota = lax.iota(jnp.int32, 16)
        zf = jnp.zeros((16,), jnp.float32)

        pltpu.sync_copy(es_hbm.at[wid], ies)
        pltpu.sync_copy(ed_hbm.at[wid], ied)
        pltpu.sync_copy(ew_hbm.at[wid], iew)
        pltpu.sync_copy(dinv_hbm, dib)
        pltpu.sync_copy(en_hbm.at[wid], env)
        ne = env[pl.ds(0, 16)][0]
        nv16 = (ne + 15) // 16

        def zbody2(i, _):
            for j in range(D // 16):
                acc[i, pl.ds(j * 16, 16)] = zf
            return 0

        lax.fori_loop(0, NDP, zbody2, 0)

        def cbody(i, _):
            s = ies[pl.ds(i * 16, 16)]
            d = ied[pl.ds(i * 16, 16)]
            w = iew[pl.ds(i * 16, 16)]
            ds_ = plsc.load_gather(dib, [s])
            dd = plsc.load_gather(dib, [d])
            iew[pl.ds(i * 16, 16)] = w * ds_ * dd
            return 0

        lax.fori_loop(0, nv16, cbody, 0)

        nct = 2 * ((ne + 2 * RB - 1) // (2 * RB))

        def fire(c, buf, sem):
            pltpu.async_copy(sup_hbm.at[ies.at[pl.ds(c * RB, RB)]], buf, sem)

        def drain(c, buf, sem):
            pltpu.make_async_copy(sup_hbm.at[ies.at[pl.ds(c * RB, RB)]], buf,
                                  sem).wait()

        zi = jnp.zeros((16,), jnp.int32)

        def process(c, buf):
            ilv = 4

            def ebody(g, _):
                eb = c * RB + g * 16
                dvec = ied[pl.ds(eb, 16)] - lo
                cvec = iew[pl.ds(eb, 16)]
                for l0 in range(0, 16, ilv):
                    cfb = [_bcast_lane(cvec, l0 + u) for u in range(ilv)]
                    dlb = [_bcast_lane(dvec, l0 + u) for u in range(ilv)]
                    for j in range(D // 16):
                        col = j * 16 + iota
                        for u in range(ilv):
                            v = cfb[u] * buf[g * 16 + l0 + u,
                                             pl.ds(j * 16, 16)]
                            plsc.addupdate_scatter(acc, [dlb[u], col], v)
                return 0

            lax.fori_loop(0, RB // 16, ebody, 0)

        @pl.when(nct > 0)
        def _():
            fire(0, rows0, sem0)

        def ch2(i, _):
            c0 = 2 * i
            fire(c0 + 1, rows1, sem1)
            drain(c0, rows0, sem0)
            process(c0, rows0)

            @pl.when(c0 + 2 < nct)
            def _():
                fire(c0 + 2, rows0, sem0)

            drain(c0 + 1, rows1, sem1)
            process(c0 + 1, rows1)
            return 0

        lax.fori_loop(0, nct // 2, ch2, 0)

        def sbody(m, _):
            pltpu.sync_copy(sup_hbm.at[pl.ds(lo + m * 16, 16)], tmps)
            if with_comb:
                pltpu.sync_copy(y1_hbm.at[pl.ds(lo + m * 16, 16)], tmp1)
            rr = lo + m * 16 + iota
            dvv = plsc.load_gather(dib, [rr])
            d2 = dvv * dvv
            for l in range(16):
                rloc = m * 16 + l
                d2l = d2[l]
                for j in range(D // 16):
                    a = acc[rloc, pl.ds(j * 16, 16)]
                    sv = tmps[l, pl.ds(j * 16, 16)]
                    out = _sigmoid(a + d2l * sv)
                    if with_comb:
                        out = GAMMA * tmp1[l, pl.ds(j * 16, 16)] + \
                            (1.0 - GAMMA) * out
                    acc[rloc, pl.ds(j * 16, 16)] = out
            return 0

        lax.fori_loop(0, NDP // 16, sbody, 0)

        pltpu.sync_copy(acc, y_hbm.at[pl.ds(lo, NDP)])

    ins = (es, ed, ew, dinv, en, sup) + ((y1,) if with_comb else ())
    f = pl.kernel(
        body,
        out_type=jax.ShapeDtypeStruct((npad, D), jnp.float32),
        mesh=_sc_mesh(),
        scratch_types=[
            pltpu.VMEM((CAP,), jnp.int32),
            pltpu.VMEM((CAP,), jnp.int32),
            pltpu.VMEM((CAP,), jnp.float32),
            pltpu.VMEM((npad,), jnp.float32),
            pltpu.VMEM((NDP, D), jnp.float32),
            pltpu.VMEM((RB, D), jnp.float32),
            pltpu.VMEM((RB, D), jnp.float32),
            pltpu.VMEM((16, D), jnp.float32),
            pltpu.VMEM((16, D), jnp.float32),
            pltpu.VMEM((16,), jnp.int32),
            pltpu.SemaphoreType.DMA,
            pltpu.SemaphoreType.DMA,
        ],
        compiler_params=pltpu.CompilerParams(needs_layout_passes=False),
    )
    return f(*ins)


def _matmul(x, w):
    bm = 1024
    npad = x.shape[0]

    def body(x_ref, w_ref, o_ref):
        o_ref[...] = jnp.dot(x_ref[...], w_ref[...],
                             preferred_element_type=jnp.float32)

    return pl.pallas_call(
        body,
        grid=(npad // bm,),
        in_specs=[
            pl.BlockSpec((bm, D), lambda i: (i, 0)),
            pl.BlockSpec((D, D), lambda i: (0, 0)),
        ],
        out_specs=pl.BlockSpec((bm, D), lambda i: (i, 0)),
        out_shape=jax.ShapeDtypeStruct((npad, D), jnp.float32),
    )(x, w)


def _gated_combine(xp, comb, wg):
    bm = 1024
    npad = xp.shape[0]

    def body(x_ref, c_ref, w_ref, o_ref):
        c = c_ref[...]
        g = jnp.dot(c, w_ref[...], preferred_element_type=jnp.float32)
        o_ref[...] = x_ref[...] + jax.nn.sigmoid(g) * c

    return pl.pallas_call(
        body,
        grid=(npad // bm,),
        in_specs=[
            pl.BlockSpec((bm, D), lambda i: (i, 0)),
            pl.BlockSpec((bm, D), lambda i: (i, 0)),
            pl.BlockSpec((D, D), lambda i: (0, 0)),
        ],
        out_specs=pl.BlockSpec((bm, D), lambda i: (i, 0)),
        out_shape=jax.ShapeDtypeStruct((npad, D), jnp.float32),
    )(xp, comb, wg)


def kernel(node_embeddings, cached_edges, weight1, weight_gate):
    nt = _num_tiles()
    npad = nt * NDP
    srcs = cached_edges[:, 0].astype(jnp.int32)
    dsts = cached_edges[:, 1].astype(jnp.int32)
    xp = jnp.zeros((npad, D), jnp.float32).at[:N].set(node_embeddings)

    es, ed, ew, dinv, en = _edge_prep(srcs, dsts)
    sup1 = _matmul(xp, weight1)
    y1 = _layer(es, ed, ew, dinv, en, sup1, None)
    sup2 = _matmul(y1, weight1)
    comb = _layer(es, ed, ew, dinv, en, sup2, y1)
    out = _gated_combine(xp, comb, weight_gate)
    return out[:N]
